# baseline (speedup 1.0000x reference)
import functools
import math

import jax
import jax.numpy as jnp
from jax import lax
from jax.experimental import pallas as pl
from jax.experimental.pallas import tpu as pltpu

_VMEM_LIMIT_BYTES = 100 * 1024 * 1024


def _fused_kernel(x_ref, w_ref, gb_ref, o_ref, conv_keep, stats_acc, affine,
                  scratch_ref, *, n_img, nq, qr, w_img, pad, eps):
    cin = x_ref.shape[1]
    rows = nq * qr
    p = w_ref.shape[1]
    i = pl.program_id(0)

    @pl.when(i == 0)
    def _init():
        stats_acc[...] = jnp.zeros((2, p), jnp.float32)

    @pl.when(i < n_img)
    def _conv_phase():
        xt = jnp.transpose(x_ref[0], (1, 0))
        r = lax.broadcasted_iota(jnp.int32, (rows, 1), 0) % w_img
        scratch_ref[pl.ds(0, pad), :] = jnp.zeros((pad, 3 * cin), jnp.float32)
        scratch_ref[pl.ds(pad + rows, pad), :] = jnp.zeros((pad, 3 * cin),
                                                           jnp.float32)
        scratch_ref[pl.ds(pad, rows), 0:cin] = jnp.where(
            r != 0, jnp.roll(xt, 1, axis=0), 0.0)
        scratch_ref[pl.ds(pad, rows), cin:2 * cin] = xt
        scratch_ref[pl.ds(pad, rows), 2 * cin:3 * cin] = jnp.where(
            r != (w_img - 1), jnp.roll(xt, -1, axis=0), 0.0)

        pieces = []
        for q in range(nq):
            base = pad + q * qr
            for dh in (-1, 0, 1):
                pieces.append(scratch_ref[pl.ds(base + dh * w_img, qr), :])
        patches = jnp.concatenate(pieces, axis=1)
        acc = jnp.dot(patches, w_ref[...],
                      preferred_element_type=jnp.float32)
        conv_keep[pl.ds(i, 1)] = acc[None].astype(conv_keep.dtype)
        s1 = jnp.sum(acc, axis=0, keepdims=True)
        s2 = jnp.sum(acc * acc, axis=0, keepdims=True)
        stats_acc[...] += jnp.concatenate([s1, s2], axis=0)

    @pl.when(i == n_img)
    def _fold_phase():
        st = stats_acc[...]
        cout = p // nq
        tot = st
        for k in range(1, nq):
            tot = tot + jnp.roll(st, k * cout, axis=1)
        count = jnp.float32(n_img * rows)
        mean = tot[0:1, :] / count
        var = jnp.maximum(tot[1:2, :] / count - mean * mean, 0.0)
        scale = gb_ref[0:1, :] * lax.rsqrt(var + eps)
        shift = gb_ref[1:2, :] - mean * scale
        affine[...] = jnp.concatenate([scale, shift], axis=0)

    @pl.when(i > n_img)
    def _bn_phase():
        j = i - n_img - 1
        cv = conv_keep[pl.ds(j, 1)][0].astype(jnp.float32)
        y = cv * affine[0:1, :] + affine[1:2, :]
        y = jnp.maximum(y, 0.0)
        yt = jnp.transpose(y, (1, 0))
        ytr = yt.reshape(nq, -1, qr)
        for q in range(nq):
            o_ref[0, :, pl.ds(q * qr, qr)] = ytr[q]


@jax.jit
def _conv_block(x_nchw, weight_oihw, gamma, beta):
    N, Cin, H, W = x_nchw.shape
    Cout, _, KH, KW = weight_oihw.shape
    rows = H * W
    Q = 4 if rows % 4 == 0 else 1
    qr = rows // Q
    pad = ((W + 7) // 8) * 8
    P = Q * Cout

    x_r = x_nchw.reshape(N, Cin, rows)

    w_mat = jnp.transpose(weight_oihw, (2, 3, 1, 0)).astype(jnp.float32)
    w_mat = w_mat.reshape(KH * KW * Cin, Cout)
    w_big = jnp.kron(jnp.eye(Q, dtype=jnp.float32), w_mat)
    gb = jnp.concatenate([jnp.tile(gamma.astype(jnp.float32), Q)[None],
                          jnp.tile(beta.astype(jnp.float32), Q)[None]], axis=0)

    cparams = pltpu.CompilerParams(dimension_semantics=("arbitrary",),
                                   vmem_limit_bytes=_VMEM_LIMIT_BYTES)

    body = functools.partial(_fused_kernel, n_img=N, nq=Q, qr=qr, w_img=W,
                             pad=pad, eps=1e-5)
    out = pl.pallas_call(
        body,
        out_shape=jax.ShapeDtypeStruct((N, Cout, rows), jnp.float32),
        grid=(2 * N + 1,),
        in_specs=[pl.BlockSpec((1, Cin, rows),
                               lambda i: (jnp.minimum(i, N - 1), 0, 0)),
                  pl.BlockSpec((Q * KH * KW * Cin, P), lambda i: (0, 0)),
                  pl.BlockSpec((2, P), lambda i: (0, 0))],
        out_specs=pl.BlockSpec((1, Cout, rows),
                               lambda i: (jnp.maximum(i - N - 1, 0), 0, 0)),
        scratch_shapes=[pltpu.VMEM((N, qr, P), jnp.bfloat16),
                        pltpu.VMEM((2, P), jnp.float32),
                        pltpu.VMEM((2, P), jnp.float32),
                        pltpu.VMEM((rows + 2 * pad, 3 * Cin), jnp.float32)],
        compiler_params=cparams,
    )(x_r, w_big, gb)

    return out.reshape(N, Cout, H, W)


def kernel(x_nchw, weight_oihw, gamma, beta):
    return _conv_block(x_nchw, weight_oihw, gamma, beta)


# speedup vs baseline: 1.0591x; 1.0591x over previous
import functools
import math

import jax
import jax.numpy as jnp
from jax import lax
from jax.experimental import pallas as pl
from jax.experimental.pallas import tpu as pltpu

_VMEM_LIMIT_BYTES = 100 * 1024 * 1024


def _fused_kernel(x_ref, w_ref, gb_ref, o_ref, conv_keep, stats_acc, affine,
                  scratch_ref, *, n_img, pair, nq, qr, w_img, pad, eps):
    cin = x_ref.shape[1]
    rows = nq * qr
    p = w_ref.shape[1]
    n_steps = n_img // pair
    i = pl.program_id(0)

    @pl.when(i == 0)
    def _init():
        stats_acc[...] = jnp.zeros((2, p), jnp.float32)

    @pl.when(i < n_steps)
    def _conv_phase():
        r = lax.broadcasted_iota(jnp.int32, (qr, 1), 0) % w_img
        mask_l = r != 0
        mask_r = r != (w_img - 1)
        scratch_ref[pl.ds(0, pad), :] = jnp.zeros((pad, cin), jnp.float32)
        scratch_ref[pl.ds(pad + rows, pad), :] = jnp.zeros((pad, cin),
                                                           jnp.float32)
        for s in range(pair):
            xt = jnp.transpose(x_ref[s], (1, 0))
            scratch_ref[pl.ds(pad, rows), :] = xt
            pieces = []
            for q in range(nq):
                base = pad + q * qr
                for dh in (-1, 0, 1):
                    for dw in (-1, 0, 1):
                        sl = scratch_ref[pl.ds(base + dh * w_img + dw, qr), :]
                        if dw == -1:
                            sl = jnp.where(mask_l, sl, 0.0)
                        elif dw == 1:
                            sl = jnp.where(mask_r, sl, 0.0)
                        pieces.append(sl)
            patches = jnp.concatenate(pieces, axis=1)
            acc = jnp.dot(patches, w_ref[...],
                          preferred_element_type=jnp.float32)
            conv_keep[pl.ds(i * pair + s, 1)] = acc[None].astype(conv_keep.dtype)
            s1 = jnp.sum(acc, axis=0, keepdims=True)
            s2 = jnp.sum(acc * acc, axis=0, keepdims=True)
            stats_acc[...] += jnp.concatenate([s1, s2], axis=0)

    @pl.when(i == n_steps)
    def _fold_phase():
        st = stats_acc[...]
        cout = p // nq
        tot = st
        for k in range(1, nq):
            tot = tot + jnp.roll(st, k * cout, axis=1)
        count = jnp.float32(n_img * rows)
        mean = tot[0:1, :] / count
        var = jnp.maximum(tot[1:2, :] / count - mean * mean, 0.0)
        scale = gb_ref[0:1, :] * lax.rsqrt(var + eps)
        shift = gb_ref[1:2, :] - mean * scale
        affine[...] = jnp.concatenate([scale, shift], axis=0)

    @pl.when(i > n_steps)
    def _bn_phase():
        j = i - n_steps - 1
        for s in range(pair):
            cv = conv_keep[pl.ds(j * pair + s, 1)][0].astype(jnp.float32)
            y = cv * affine[0:1, :] + affine[1:2, :]
            y = jnp.maximum(y, 0.0)
            yt = jnp.transpose(y, (1, 0))
            ytr = yt.reshape(nq, -1, qr)
            for q in range(nq):
                o_ref[s, :, pl.ds(q * qr, qr)] = ytr[q]


@jax.jit
def _conv_block(x_nchw, weight_oihw, gamma, beta):
    N, Cin, H, W = x_nchw.shape
    Cout, _, KH, KW = weight_oihw.shape
    rows = H * W
    Q = 4 if rows % 4 == 0 else 1
    qr = rows // Q
    pad = W + 1
    P = Q * Cout

    x_r = x_nchw.reshape(N, Cin, rows)

    w_mat = jnp.transpose(weight_oihw, (2, 3, 1, 0)).astype(jnp.float32)
    w_mat = w_mat.reshape(KH * KW * Cin, Cout)
    w_big = jnp.kron(jnp.eye(Q, dtype=jnp.float32), w_mat)
    gb = jnp.concatenate([jnp.tile(gamma.astype(jnp.float32), Q)[None],
                          jnp.tile(beta.astype(jnp.float32), Q)[None]], axis=0)

    cparams = pltpu.CompilerParams(dimension_semantics=("arbitrary",),
                                   vmem_limit_bytes=_VMEM_LIMIT_BYTES)

    PAIR = 2 if N % 2 == 0 else 1
    NS = N // PAIR
    body = functools.partial(_fused_kernel, n_img=N, pair=PAIR, nq=Q, qr=qr,
                             w_img=W, pad=pad, eps=1e-5)
    out = pl.pallas_call(
        body,
        out_shape=jax.ShapeDtypeStruct((N, Cout, rows), jnp.float32),
        grid=(2 * NS + 1,),
        in_specs=[pl.BlockSpec((PAIR, Cin, rows),
                               lambda i: (jnp.minimum(i, NS - 1), 0, 0)),
                  pl.BlockSpec((Q * KH * KW * Cin, P), lambda i: (0, 0)),
                  pl.BlockSpec((2, P), lambda i: (0, 0))],
        out_specs=pl.BlockSpec((PAIR, Cout, rows),
                               lambda i: (jnp.maximum(i - NS - 1, 0), 0, 0)),
        scratch_shapes=[pltpu.VMEM((N, qr, P), jnp.bfloat16),
                        pltpu.VMEM((2, P), jnp.float32),
                        pltpu.VMEM((2, P), jnp.float32),
                        pltpu.VMEM((rows + 2 * pad, Cin), jnp.float32)],
        compiler_params=cparams,
    )(x_r, w_big, gb)

    return out.reshape(N, Cout, H, W)


def kernel(x_nchw, weight_oihw, gamma, beta):
    return _conv_block(x_nchw, weight_oihw, gamma, beta)
